# baseline (speedup 1.0000x reference)
import functools

import jax
import jax.numpy as jnp
from jax import lax
from jax.experimental import pallas as pl
from jax.experimental.pallas import tpu as pltpu
from jax.experimental.pallas import tpu_sc as plsc

B, E, N, D = 2, 160000, 10000, 128
NS = 16
EPT = E // NS
CHUNK = 128
SUB = 128
NCHUNK = EPT // CHUNK
TAIL = EPT - NCHUNK * CHUNK
EPT_PAD = 10240
IROWS = EPT_PAD // SUB
ROWS_OUT = 624
ROWS_REM = N - NS * ROWS_OUT


def _make_scatter_kernel():
    mesh = plsc.VectorSubcoreMesh(core_axis_name="c", subcore_axis_name="s")

    @functools.partial(
        pl.kernel,
        mesh=mesh,
        out_type=jax.ShapeDtypeStruct((B, N, D), jnp.float32),
        scratch_types=[
            pltpu.VMEM((CHUNK, D), jnp.float32),
            pltpu.VMEM((CHUNK, D), jnp.float32),
            pltpu.VMEM((IROWS, SUB), jnp.int32),
            pltpu.VMEM((16,), jnp.int32),
            pltpu.VMEM_SHARED((N, D), jnp.float32),
            pltpu.SemaphoreType.DMA,
            pltpu.SemaphoreType.DMA,
            pltpu.SemaphoreType.DMA,
            pltpu.SemaphoreType.DMA,
        ],
    )
    def scatter_kernel(msg_hbm, idx_hbm, out_hbm,
                       msg_v0, msg_v1, idx_v, tidx_v, acc_sh,
                       sem_in0, sem_in1, sem_sc0, sem_sc1):
        c = lax.axis_index("c")
        s = lax.axis_index("s")
        base = s * EPT
        bufs = ((msg_v0, sem_in0, sem_sc0), (msg_v1, sem_in1, sem_sc1))

        def issue_in(k, b):
            off = pl.multiple_of(base + k * CHUNK, 8)
            msg_v, sem, _ = bufs[b]
            pltpu.async_copy(msg_hbm.at[c, pl.ds(off, CHUNK)], msg_v, sem)

        def wait_in(b):
            msg_v, sem, _ = bufs[b]
            pltpu.make_async_copy(msg_hbm.at[c, pl.ds(0, CHUNK)], msg_v, sem).wait()

        def issue_scatter(k, b):
            msg_v, _, sem = bufs[b]
            return pltpu.async_copy(
                msg_v, acc_sh.at[idx_v.at[k]], sem, add=True
            )

        issue_in(0, 0)
        pltpu.async_copy(idx_hbm.at[c, s], idx_v, sem_in1)
        zero16 = jnp.zeros((16,), jnp.float32)

        def zrow(r, _):
            for j in range(D // 16):
                msg_v1[r, pl.ds(j * 16, 16)] = zero16
            return _

        lax.fori_loop(0, CHUNK, zrow, None)
        pltpu.make_async_copy(idx_hbm.at[c, s], idx_v, sem_in1).wait()
        tidx_v[pl.ds(0, 16)] = idx_v[NCHUNK, pl.ds(0, 16)]

        zbase = pl.multiple_of(s * ROWS_OUT, 8)
        zhandles = []
        for q in range(ROWS_OUT // CHUNK):
            zhandles.append(pltpu.async_copy(
                msg_v1.at[pl.ds(0, CHUNK)],
                acc_sh.at[pl.ds(zbase + q * CHUNK, CHUNK)],
                sem_sc0,
            ))
        ztail = ROWS_OUT % CHUNK
        if ztail:
            zhandles.append(pltpu.async_copy(
                msg_v1.at[pl.ds(0, ztail)],
                acc_sh.at[pl.ds(zbase + ROWS_OUT - ztail, ztail)],
                sem_sc0,
            ))

        @pl.when(s == NS - 1)
        def _():
            pltpu.async_copy(
                msg_v1.at[pl.ds(0, ROWS_REM)],
                acc_sh.at[pl.ds(NS * ROWS_OUT, ROWS_REM)],
                sem_sc0,
            ).wait()

        for h in zhandles:
            h.wait()

        issue_in(1, 1)
        plsc.subcore_barrier()

        def body(t, _):
            k0 = t * 2
            k1 = t * 2 + 1
            wait_in(0)
            h0 = issue_scatter(k0, 0)
            wait_in(1)
            h1 = issue_scatter(k1, 1)
            h0.wait()

            @pl.when(k0 + 2 < NCHUNK)
            def _():
                issue_in(k0 + 2, 0)

            h1.wait()

            @pl.when(k1 + 2 < NCHUNK)
            def _():
                issue_in(k1 + 2, 1)
            return _

        lax.fori_loop(0, NCHUNK // 2, body, None)

        toff = pl.multiple_of(base + NCHUNK * CHUNK, 8)
        pltpu.async_copy(
            msg_hbm.at[c, pl.ds(toff, TAIL)], msg_v0.at[pl.ds(0, TAIL)], sem_in0
        )
        pltpu.make_async_copy(
            msg_hbm.at[c, pl.ds(0, TAIL)], msg_v0.at[pl.ds(0, TAIL)], sem_in0
        ).wait()
        pltpu.async_copy(
            msg_v0.at[pl.ds(0, TAIL)],
            acc_sh.at[tidx_v],
            sem_sc0,
            add=True,
        ).wait()

        plsc.subcore_barrier()

        obase = pl.multiple_of(s * ROWS_OUT, 8)
        pltpu.sync_copy(
            acc_sh.at[pl.ds(obase, ROWS_OUT)],
            out_hbm.at[c, pl.ds(obase, ROWS_OUT)],
        )

        @pl.when(s == NS - 1)
        def _():
            pltpu.sync_copy(
                acc_sh.at[pl.ds(NS * ROWS_OUT, ROWS_REM)],
                out_hbm.at[c, pl.ds(NS * ROWS_OUT, ROWS_REM)],
            )

    return scatter_kernel


_scatter = _make_scatter_kernel()


def kernel(messages, tgt_indices, atom_features_ref):
    del atom_features_ref
    idx4 = jnp.pad(
        tgt_indices.reshape(B, NS, EPT), ((0, 0), (0, 0), (0, EPT_PAD - EPT))
    ).reshape(B, NS, IROWS, SUB)
    return _scatter(messages, idx4)
